# baseline (speedup 1.0000x reference)
import functools

import jax
import jax.numpy as jnp
import numpy as np
from jax import lax
from jax.experimental import pallas as pl
from jax.experimental.pallas import tpu as pltpu
from jax.experimental.pallas import tpu_sc as plsc

_N = 10000
_D = 128
_H = 8
_C1 = 8
_C2 = 128
NPAD = 10112
PADE = 331776
EPT = PADE // 32
ZR = NPAD // 16

_mesh = plsc.VectorSubcoreMesh(core_axis_name="c", subcore_axis_name="s")
_sc_params = pltpu.CompilerParams(needs_layout_passes=False,
                                  use_tc_tiling_on_sc=False)


def _tc_a_body(x_ref, w_ref, o_ref):
    o_ref[...] = jnp.dot(x_ref[...], w_ref[...], preferred_element_type=jnp.float32)


def _tc_a(xp, wcat1):
    return pl.pallas_call(
        _tc_a_body,
        out_shape=jax.ShapeDtypeStruct((NPAD, 80), jnp.float32),
    )(xp, wcat1)


@functools.partial(
    pl.kernel,
    mesh=_mesh,
    compiler_params=_sc_params,
    out_type=jax.ShapeDtypeStruct((2, NPAD, 80), jnp.float32),
    scratch_types=[
        pltpu.VMEM((128,), jnp.int32),
        pltpu.VMEM((128,), jnp.int32),
        pltpu.VMEM((128, 80), jnp.float32),
        pltpu.VMEM((128, 80), jnp.float32),
        pltpu.VMEM((128, 80), jnp.float32),
        pltpu.VMEM_SHARED((NPAD, 80), jnp.float32),
        pltpu.SemaphoreType.DMA,
        pltpu.SemaphoreType.DMA,
    ],
)
def _sc_l1(rows1, src_i, dst_i, zeros80, out, sidx, didx, srcb, dstb, wb, acc,
           sem1, sem2):
    c = lax.axis_index("c")
    s = lax.axis_index("s")
    tid = c * 16 + s

    pltpu.sync_copy(zeros80.at[pl.ds(s * ZR, ZR)], acc.at[pl.ds(s * ZR, ZR)])
    plsc.subcore_barrier()

    lanes = lax.iota(jnp.int32, 16)
    hl = lanes & 7
    eoff = lanes >> 3

    def block(b, carry):
        ebase = tid * EPT + b * 128
        pltpu.sync_copy(src_i.at[pl.ds(ebase, 128)], sidx)
        pltpu.sync_copy(dst_i.at[pl.ds(ebase, 128)], didx)
        cp1 = pltpu.async_copy(rows1.at[sidx], srcb, sem1)
        cp2 = pltpu.async_copy(rows1.at[didx], dstb, sem2)
        cp1.wait()
        cp2.wait()

        def pair(p, carry2):
            row = 2 * p + eoff
            accv = jnp.zeros((16,), jnp.float32)
            svs = []
            for cc in range(8):
                col = hl * 8 + cc
                sv = plsc.load_gather(srcb, [row, col])
                dv = plsc.load_gather(dstb, [row, col])
                svs.append(sv)
                accv = accv + sv * dv
            als = plsc.load_gather(srcb, [row, hl + 64])
            ard = plsc.load_gather(dstb, [row, hl + 72])
            a = (als + ard) / (1.0 + jnp.exp(-accv))
            a = jnp.where(a >= 0.0, a, 0.2 * a)
            w = jnp.exp(a)
            for cc in range(8):
                plsc.store_scatter(wb, [row, hl * 8 + cc], w * svs[cc])
            plsc.store_scatter(wb, [row, hl + 64], w)
            return carry2

        lax.fori_loop(0, 64, pair, 0)
        pltpu.sync_copy(wb, acc.at[didx], add=True)
        return carry

    lax.fori_loop(0, EPT // 128, block, 0)
    plsc.subcore_barrier()
    pltpu.sync_copy(acc.at[pl.ds(s * ZR, ZR)], out.at[c, pl.ds(s * ZR, ZR)])


def _tc_b_body(p_ref, w2s_ref, w2_ref, wlr_ref, b1_ref, big1_ref, big2_ref,
               alar_ref):
    p = p_ref[0] + p_ref[1]
    num = p[:, :64]
    deni = 1.0 / (p[:, 64:72] + 1e-16)
    y = (num.reshape(num.shape[0], 8, 8) * deni[:, :, None]).reshape(num.shape[0], 64)
    y = y + b1_ref[...]
    y = jnp.where(y > 0.0, y, jnp.exp(jnp.minimum(y, 0.0)) - 1.0)
    big1_ref[...] = jnp.dot(y, w2s_ref[...], preferred_element_type=jnp.float32)
    big2_ref[...] = jnp.dot(y, w2_ref[...], preferred_element_type=jnp.float32)
    alar_ref[...] = jnp.dot(y, wlr_ref[...], preferred_element_type=jnp.float32)


def _tc_b(parts1, w2s, w2, wlr2, b1r):
    rb = NPAD // 8
    return pl.pallas_call(
        _tc_b_body,
        grid=(8,),
        in_specs=[
            pl.BlockSpec((2, rb, 80), lambda i: (0, i, 0)),
            pl.BlockSpec((64, 1024), lambda i: (0, 0)),
            pl.BlockSpec((64, 1024), lambda i: (0, 0)),
            pl.BlockSpec((64, 16), lambda i: (0, 0)),
            pl.BlockSpec((1, 64), lambda i: (0, 0)),
        ],
        out_specs=[
            pl.BlockSpec((rb, 1024), lambda i: (i, 0)),
            pl.BlockSpec((rb, 1024), lambda i: (i, 0)),
            pl.BlockSpec((rb, 16), lambda i: (i, 0)),
        ],
        out_shape=[
            jax.ShapeDtypeStruct((NPAD, 1024), jnp.float32),
            jax.ShapeDtypeStruct((NPAD, 1024), jnp.float32),
            jax.ShapeDtypeStruct((NPAD, 16), jnp.float32),
        ],
    )(parts1, w2s, w2, wlr2, b1r)


EB1 = 16
NB1 = EPT // EB1


@functools.partial(
    pl.kernel,
    mesh=_mesh,
    compiler_params=_sc_params,
    out_type=(
        jax.ShapeDtypeStruct((PADE, 8), jnp.float32),
        jax.ShapeDtypeStruct((2, NPAD, 8), jnp.float32),
    ),
    scratch_types=[
        pltpu.VMEM((NB1, EB1), jnp.int32),
        pltpu.VMEM((NB1, EB1), jnp.int32),
        pltpu.VMEM((2, EB1, 1024), jnp.float32),
        pltpu.VMEM((2, EB1, 1024), jnp.float32),
        pltpu.VMEM((2, EB1, 16), jnp.float32),
        pltpu.VMEM((2, EB1, 16), jnp.float32),
        pltpu.VMEM((2, EB1, 8), jnp.float32),
        pltpu.VMEM_SHARED((NPAD, 8), jnp.float32),
        pltpu.SemaphoreType.DMA,
        pltpu.SemaphoreType.DMA,
        pltpu.SemaphoreType.DMA,
        pltpu.SemaphoreType.DMA,
        pltpu.SemaphoreType.DMA,
        pltpu.SemaphoreType.DMA,
        pltpu.SemaphoreType.DMA,
        pltpu.SemaphoreType.DMA,
    ],
)
def _sc_p1(big, alar2, src_i2, dst_i2, zeros8, w_out, den_out, sidx2, didx2,
           srcb, dstb, sala, dala, wb, den, s0, s1, s2, s3, s4, s5, s6, s7):
    c = lax.axis_index("c")
    s = lax.axis_index("s")
    tid = c * 16 + s
    sems = ((s0, s1, s2, s3), (s4, s5, s6, s7))

    pltpu.sync_copy(zeros8.at[pl.ds(s * ZR, ZR)], den.at[pl.ds(s * ZR, ZR)])
    pltpu.sync_copy(src_i2.at[pl.ds(tid * NB1, NB1)], sidx2)
    pltpu.sync_copy(dst_i2.at[pl.ds(tid * NB1, NB1)], didx2)
    plsc.subcore_barrier()

    lanes = lax.iota(jnp.int32, 16)
    hl = lanes & 7
    eoff = lanes >> 3
    hl8 = hl * 8

    def copies(u, b):
        return [
            pltpu.make_async_copy(big.at[sidx2.at[b]], srcb.at[u], sems[u][0]),
            pltpu.make_async_copy(big.at[didx2.at[b]], dstb.at[u], sems[u][1]),
            pltpu.make_async_copy(alar2.at[sidx2.at[b]], sala.at[u], sems[u][2]),
            pltpu.make_async_copy(alar2.at[didx2.at[b]], dala.at[u], sems[u][3]),
        ]

    def issue(u, b):
        for cp in copies(u, b):
            cp.start()

    def wait(u, b):
        for cp in copies(u, b):
            cp.wait()

    def compute(u, b):
        def pair(p, carry2):
            row = 2 * p + eoff
            accs = [jnp.zeros((16,), jnp.float32) for _ in range(4)]
            for f in range(16):
                for cc in range(8):
                    col = hl8 + (f * 64 + cc)
                    sv = plsc.load_gather(srcb.at[u], [row, col])
                    dv = plsc.load_gather(dstb.at[u], [row, col])
                    accs[cc & 3] = accs[cc & 3] + sv * dv
            accv = (accs[0] + accs[1]) + (accs[2] + accs[3])
            als = plsc.load_gather(sala.at[u], [row, hl])
            ard = plsc.load_gather(dala.at[u], [row, hl + 8])
            a = (als + ard) / (1.0 + jnp.exp(-accv))
            a = jnp.where(a >= 0.0, a, 0.2 * a)
            w = jnp.exp(a)
            plsc.store_scatter(wb.at[u], [row, hl], w)
            return carry2

        lax.fori_loop(0, EB1 // 2, pair, 0)
        pltpu.sync_copy(wb.at[u], w_out.at[pl.ds(tid * EPT + b * EB1, EB1)])
        pltpu.sync_copy(wb.at[u], den.at[didx2.at[b]], add=True)

    issue(0, 0)

    def step(i, carry):
        b0 = 2 * i
        b1 = b0 + 1
        b2 = jnp.minimum(b0 + 2, NB1 - 1)
        issue(1, b1)
        wait(0, b0)
        compute(0, b0)
        issue(0, b2)
        wait(1, b1)
        compute(1, b1)
        return carry

    lax.fori_loop(0, NB1 // 2, step, 0)
    wait(0, NB1 - 1)
    plsc.subcore_barrier()
    pltpu.sync_copy(den.at[pl.ds(s * ZR, ZR)], den_out.at[c, pl.ds(s * ZR, ZR)])


EB2 = 128
NB2 = PADE // 16 // EB2
CW = 64


@functools.partial(
    pl.kernel,
    mesh=_mesh,
    compiler_params=_sc_params,
    out_type=jax.ShapeDtypeStruct((16, NPAD, CW), jnp.float32),
    scratch_types=[
        pltpu.VMEM((NB2, EB2), jnp.int32),
        pltpu.VMEM((NB2, EB2), jnp.int32),
        pltpu.VMEM((2, EB2, CW), jnp.float32),
        pltpu.VMEM((2, EB2, 8), jnp.float32),
        pltpu.VMEM((2, EB2, CW), jnp.float32),
        pltpu.VMEM_SHARED((NPAD, CW), jnp.float32),
        pltpu.SemaphoreType.DMA,
        pltpu.SemaphoreType.DMA,
        pltpu.SemaphoreType.DMA,
        pltpu.SemaphoreType.DMA,
    ],
)
def _sc_p2(big_r, w_e, src16_2, dst_i2, zeros_cw, out, gidx2, didx2, xb, wb,
           yb, num, g0, g1, w0, w1):
    c = lax.axis_index("c")
    s = lax.axis_index("s")
    gsems = (g0, g1)
    wsems = (w0, w1)

    lanes = lax.iota(jnp.int32, 16)

    pltpu.sync_copy(dst_i2.at[pl.ds(s * NB2, NB2)], didx2)

    for j in range(8):
        fidx = c * 8 + j
        pltpu.sync_copy(zeros_cw.at[pl.ds(s * ZR, ZR)], num.at[pl.ds(s * ZR, ZR)])
        pltpu.sync_copy(src16_2.at[fidx, pl.ds(s * NB2, NB2)], gidx2)
        plsc.subcore_barrier()

        def copies(u, b):
            ebase = s * (PADE // 16) + b * EB2
            return [
                pltpu.make_async_copy(big_r.at[gidx2.at[b]], xb.at[u], gsems[u]),
                pltpu.make_async_copy(w_e.at[pl.ds(ebase, EB2)], wb.at[u], wsems[u]),
            ]

        def issue(u, b):
            for cp in copies(u, b):
                cp.start()

        def wait(u, b):
            for cp in copies(u, b):
                cp.wait()

        hvec = jnp.full((16,), fidx >> 1, jnp.int32)

        def compute(u, b):
            def edge(e2, carry2):
                for d in range(2):
                    e = 2 * e2 + d
                    ev = jnp.full((16,), e, jnp.int32)
                    wv = plsc.load_gather(wb.at[u], [ev, hvec])
                    for k in range(CW // 16):
                        yb.at[u][e, pl.ds(k * 16, 16)] = (
                            xb.at[u][e, pl.ds(k * 16, 16)] * wv)
                return carry2

            lax.fori_loop(0, EB2 // 2, edge, 0)
            pltpu.sync_copy(yb.at[u], num.at[didx2.at[b]], add=True)

        issue(0, 0)

        def step(i, carry):
            b0 = 2 * i
            b1 = b0 + 1
            b2 = jnp.minimum(b0 + 2, NB2 - 1)
            issue(1, b1)
            wait(0, b0)
            compute(0, b0)
            issue(0, b2)
            wait(1, b1)
            compute(1, b1)
            return carry

        lax.fori_loop(0, NB2 // 2, step, 0)
        wait(0, NB2 - 1)
        plsc.subcore_barrier()
        pltpu.sync_copy(num.at[pl.ds(s * ZR, ZR)], out.at[fidx, pl.ds(s * ZR, ZR)])
        plsc.subcore_barrier()


def _tc_c_body(np_ref, dp_ref, b2_ref, o_ref):
    deni = 1.0 / (dp_ref[0] + dp_ref[1] + 1e-16)
    rb = deni.shape[0]
    acc = jnp.zeros((rb, 128), jnp.float32)
    for h in range(8):
        part = jnp.concatenate([np_ref[2 * h], np_ref[2 * h + 1]], axis=1)
        acc = acc + part * deni[:, h:h + 1]
    o = acc * 0.125 + b2_ref[...]
    m = jnp.max(o, axis=-1, keepdims=True)
    o_ref[...] = o - (m + jnp.log(jnp.sum(jnp.exp(o - m), axis=-1, keepdims=True)))


def _tc_c(numparts, denparts, b2r):
    rb = NPAD // 16
    return pl.pallas_call(
        _tc_c_body,
        grid=(16,),
        in_specs=[
            pl.BlockSpec((16, rb, CW), lambda i: (0, i, 0)),
            pl.BlockSpec((2, rb, 8), lambda i: (0, i, 0)),
            pl.BlockSpec((1, 128), lambda i: (0, 0)),
        ],
        out_specs=pl.BlockSpec((rb, 128), lambda i: (i, 0)),
        out_shape=jax.ShapeDtypeStruct((NPAD, 128), jnp.float32),
    )(numparts, denparts, b2r)


def kernel(x, edge_index, W1, att_l1, att_r1, b1, W2, att_l2, att_r2, b2):
    ei = edge_index.astype(jnp.int32)
    loop = jnp.arange(_N, dtype=jnp.int32)
    src = jnp.concatenate([ei[0], loop])
    dst = jnp.concatenate([ei[1], loop])
    npad_e = PADE - src.shape[0]
    src = jnp.concatenate([src, jnp.full((npad_e,), _N, jnp.int32)])
    dst = jnp.concatenate([dst, jnp.full((npad_e,), _N, jnp.int32)])

    w1r = W1.reshape(_D, _H, _C1)
    wcat1 = jnp.concatenate(
        [W1,
         jnp.einsum("dhc,hc->dh", w1r, att_l1[0]),
         jnp.einsum("dhc,hc->dh", w1r, att_r1[0])], axis=1)

    jcol = np.arange(_H * _C2)
    fj, hj, ccj = jcol // 64, (jcol % 64) // 8, jcol % 8
    w2s = W2[:, hj * _C2 + fj * 8 + ccj]
    w2r = W2.reshape(_H * _C1, _H, _C2)
    wlr2 = jnp.concatenate(
        [jnp.einsum("dhc,hc->dh", w2r, att_l2[0]),
         jnp.einsum("dhc,hc->dh", w2r, att_r2[0])], axis=1)

    xp = jnp.pad(x, ((0, NPAD - _N), (0, 0)))

    rows1 = _tc_a(xp, wcat1)
    parts1 = _sc_l1(rows1, src, dst, jnp.zeros((NPAD, 80), jnp.float32))
    big1, big2, alar2 = _tc_b(parts1, w2s, W2, wlr2, b1.reshape(1, 64))
    w_e, denparts = _sc_p1(big1, alar2, src.reshape(-1, EB1), dst.reshape(-1, EB1),
                           jnp.zeros((NPAD, 8), jnp.float32))
    src16_2 = (src[None, :] * 16 + jnp.arange(16, dtype=jnp.int32)[:, None]
               ).reshape(16, -1, EB2)
    numparts = _sc_p2(big2.reshape(NPAD * 16, CW), w_e, src16_2,
                      dst.reshape(-1, EB2), jnp.zeros((NPAD, CW), jnp.float32))
    out = _tc_c(numparts, denparts, b2.reshape(1, 128))
    return out[:_N], jnp.zeros(())


# speedup vs baseline: 1.2242x; 1.2242x over previous
import functools

import jax
import jax.numpy as jnp
import numpy as np
from jax import lax
from jax.experimental import pallas as pl
from jax.experimental.pallas import tpu as pltpu
from jax.experimental.pallas import tpu_sc as plsc

_N = 10000
_D = 128
_H = 8
_C1 = 8
_C2 = 128
NPAD = 10112
PADE = 331776
EPT = PADE // 32
ZR = NPAD // 16

_mesh = plsc.VectorSubcoreMesh(core_axis_name="c", subcore_axis_name="s")
_sc_params = pltpu.CompilerParams(needs_layout_passes=False,
                                  use_tc_tiling_on_sc=False)


def _tc_a_body(x_ref, w_ref, o_ref):
    o_ref[...] = jnp.dot(x_ref[...], w_ref[...], preferred_element_type=jnp.float32)


def _tc_a(xp, wcat1):
    return pl.pallas_call(
        _tc_a_body,
        out_shape=jax.ShapeDtypeStruct((NPAD, 80), jnp.float32),
    )(xp, wcat1)


@functools.partial(
    pl.kernel,
    mesh=_mesh,
    compiler_params=_sc_params,
    out_type=jax.ShapeDtypeStruct((2, NPAD, 80), jnp.float32),
    scratch_types=[
        pltpu.VMEM((128,), jnp.int32),
        pltpu.VMEM((128,), jnp.int32),
        pltpu.VMEM((128, 80), jnp.float32),
        pltpu.VMEM((128, 80), jnp.float32),
        pltpu.VMEM((128, 80), jnp.float32),
        pltpu.VMEM_SHARED((NPAD, 80), jnp.float32),
        pltpu.SemaphoreType.DMA,
        pltpu.SemaphoreType.DMA,
    ],
)
def _sc_l1(rows1, src_i, dst_i, zeros80, out, sidx, didx, srcb, dstb, wb, acc,
           sem1, sem2):
    c = lax.axis_index("c")
    s = lax.axis_index("s")
    tid = c * 16 + s

    pltpu.sync_copy(zeros80.at[pl.ds(s * ZR, ZR)], acc.at[pl.ds(s * ZR, ZR)])
    plsc.subcore_barrier()

    lanes = lax.iota(jnp.int32, 16)
    hl = lanes & 7
    eoff = lanes >> 3

    def block(b, carry):
        ebase = tid * EPT + b * 128
        pltpu.sync_copy(src_i.at[pl.ds(ebase, 128)], sidx)
        pltpu.sync_copy(dst_i.at[pl.ds(ebase, 128)], didx)
        cp1 = pltpu.async_copy(rows1.at[sidx], srcb, sem1)
        cp2 = pltpu.async_copy(rows1.at[didx], dstb, sem2)
        cp1.wait()
        cp2.wait()

        def pair(p, carry2):
            row = 2 * p + eoff
            accv = jnp.zeros((16,), jnp.float32)
            svs = []
            for cc in range(8):
                col = hl * 8 + cc
                sv = plsc.load_gather(srcb, [row, col])
                dv = plsc.load_gather(dstb, [row, col])
                svs.append(sv)
                accv = accv + sv * dv
            als = plsc.load_gather(srcb, [row, hl + 64])
            ard = plsc.load_gather(dstb, [row, hl + 72])
            a = (als + ard) / (1.0 + jnp.exp(-accv))
            a = jnp.where(a >= 0.0, a, 0.2 * a)
            w = jnp.exp(a)
            for cc in range(8):
                plsc.store_scatter(wb, [row, hl * 8 + cc], w * svs[cc])
            plsc.store_scatter(wb, [row, hl + 64], w)
            return carry2

        lax.fori_loop(0, 64, pair, 0)
        pltpu.sync_copy(wb, acc.at[didx], add=True)
        return carry

    lax.fori_loop(0, EPT // 128, block, 0)
    plsc.subcore_barrier()
    pltpu.sync_copy(acc.at[pl.ds(s * ZR, ZR)], out.at[c, pl.ds(s * ZR, ZR)])


def _tc_b_body(p_ref, w2s_ref, w2_ref, wlr_ref, b1_ref, big1_ref, big2_ref,
               alar_ref):
    p = p_ref[0] + p_ref[1]
    num = p[:, :64]
    deni = 1.0 / (p[:, 64:72] + 1e-16)
    y = (num.reshape(num.shape[0], 8, 8) * deni[:, :, None]).reshape(num.shape[0], 64)
    y = y + b1_ref[...]
    y = jnp.where(y > 0.0, y, jnp.exp(jnp.minimum(y, 0.0)) - 1.0)
    big1_ref[...] = jnp.dot(y, w2s_ref[...], preferred_element_type=jnp.float32)
    big2_ref[...] = jnp.dot(y, w2_ref[...], preferred_element_type=jnp.float32)
    alar_ref[...] = jnp.dot(y, wlr_ref[...], preferred_element_type=jnp.float32)


def _tc_b(parts1, w2s, w2, wlr2, b1r):
    rb = NPAD // 8
    return pl.pallas_call(
        _tc_b_body,
        grid=(8,),
        in_specs=[
            pl.BlockSpec((2, rb, 80), lambda i: (0, i, 0)),
            pl.BlockSpec((64, 1024), lambda i: (0, 0)),
            pl.BlockSpec((64, 1024), lambda i: (0, 0)),
            pl.BlockSpec((64, 16), lambda i: (0, 0)),
            pl.BlockSpec((1, 64), lambda i: (0, 0)),
        ],
        out_specs=[
            pl.BlockSpec((rb, 1024), lambda i: (i, 0)),
            pl.BlockSpec((rb, 1024), lambda i: (i, 0)),
            pl.BlockSpec((rb, 16), lambda i: (i, 0)),
        ],
        out_shape=[
            jax.ShapeDtypeStruct((NPAD, 1024), jnp.float32),
            jax.ShapeDtypeStruct((NPAD, 1024), jnp.float32),
            jax.ShapeDtypeStruct((NPAD, 16), jnp.float32),
        ],
    )(parts1, w2s, w2, wlr2, b1r)


EB1 = 16
NB1 = EPT // EB1


@functools.partial(
    pl.kernel,
    mesh=_mesh,
    compiler_params=_sc_params,
    out_type=(
        jax.ShapeDtypeStruct((PADE, 8), jnp.float32),
        jax.ShapeDtypeStruct((2, NPAD, 8), jnp.float32),
    ),
    scratch_types=[
        pltpu.VMEM((NB1, EB1), jnp.int32),
        pltpu.VMEM((NB1, EB1), jnp.int32),
        pltpu.VMEM((2, EB1, 1024), jnp.float32),
        pltpu.VMEM((2, EB1, 1024), jnp.float32),
        pltpu.VMEM((2, EB1, 16), jnp.float32),
        pltpu.VMEM((2, EB1, 16), jnp.float32),
        pltpu.VMEM((2, EB1, 8), jnp.float32),
        pltpu.VMEM_SHARED((NPAD, 8), jnp.float32),
        pltpu.SemaphoreType.DMA,
        pltpu.SemaphoreType.DMA,
        pltpu.SemaphoreType.DMA,
        pltpu.SemaphoreType.DMA,
        pltpu.SemaphoreType.DMA,
        pltpu.SemaphoreType.DMA,
        pltpu.SemaphoreType.DMA,
        pltpu.SemaphoreType.DMA,
    ],
)
def _sc_p1(big, alar2, src_i2, dst_i2, zeros8, w_out, den_out, sidx2, didx2,
           srcb, dstb, sala, dala, wb, den, s0, s1, s2, s3, s4, s5, s6, s7):
    c = lax.axis_index("c")
    s = lax.axis_index("s")
    tid = c * 16 + s
    sems = ((s0, s1, s2, s3), (s4, s5, s6, s7))

    pltpu.sync_copy(zeros8.at[pl.ds(s * ZR, ZR)], den.at[pl.ds(s * ZR, ZR)])
    pltpu.sync_copy(src_i2.at[pl.ds(tid * NB1, NB1)], sidx2)
    pltpu.sync_copy(dst_i2.at[pl.ds(tid * NB1, NB1)], didx2)
    plsc.subcore_barrier()

    lanes = lax.iota(jnp.int32, 16)
    hl = lanes & 7
    eoff = lanes >> 3
    hl8 = hl * 8

    def copies(u, b):
        return [
            pltpu.make_async_copy(big.at[sidx2.at[b]], srcb.at[u], sems[u][0]),
            pltpu.make_async_copy(big.at[didx2.at[b]], dstb.at[u], sems[u][1]),
            pltpu.make_async_copy(alar2.at[sidx2.at[b]], sala.at[u], sems[u][2]),
            pltpu.make_async_copy(alar2.at[didx2.at[b]], dala.at[u], sems[u][3]),
        ]

    def issue(u, b):
        for cp in copies(u, b):
            cp.start()

    def wait(u, b):
        for cp in copies(u, b):
            cp.wait()

    def compute(u, b):
        def pair(p, carry2):
            row = 2 * p + eoff
            accv = jnp.zeros((16,), jnp.float32)
            for f in range(16):
                for cc in range(8):
                    col = hl8 + (f * 64 + cc)
                    sv = plsc.load_gather(srcb.at[u], [row, col])
                    dv = plsc.load_gather(dstb.at[u], [row, col])
                    accv = accv + sv * dv
            als = plsc.load_gather(sala.at[u], [row, hl])
            ard = plsc.load_gather(dala.at[u], [row, hl + 8])
            a = (als + ard) / (1.0 + jnp.exp(-accv))
            a = jnp.where(a >= 0.0, a, 0.2 * a)
            w = jnp.exp(a)
            plsc.store_scatter(wb.at[u], [row, hl], w)
            return carry2

        lax.fori_loop(0, EB1 // 2, pair, 0)
        pltpu.sync_copy(wb.at[u], w_out.at[pl.ds(tid * EPT + b * EB1, EB1)])
        pltpu.sync_copy(wb.at[u], den.at[didx2.at[b]], add=True)

    issue(0, 0)

    def step(i, carry):
        b0 = 2 * i
        b1 = b0 + 1
        b2 = jnp.minimum(b0 + 2, NB1 - 1)
        issue(1, b1)
        wait(0, b0)
        compute(0, b0)
        issue(0, b2)
        wait(1, b1)
        compute(1, b1)
        return carry

    lax.fori_loop(0, NB1 // 2, step, 0)
    wait(0, NB1 - 1)
    plsc.subcore_barrier()
    pltpu.sync_copy(den.at[pl.ds(s * ZR, ZR)], den_out.at[c, pl.ds(s * ZR, ZR)])


EB2 = 128
NB2 = PADE // 16 // EB2
CW = 64


@functools.partial(
    pl.kernel,
    mesh=_mesh,
    compiler_params=_sc_params,
    out_type=jax.ShapeDtypeStruct((16, NPAD, CW), jnp.float32),
    scratch_types=[
        pltpu.VMEM((NB2, EB2), jnp.int32),
        pltpu.VMEM((NB2, EB2), jnp.int32),
        pltpu.VMEM((2, EB2, CW), jnp.float32),
        pltpu.VMEM((2, EB2, 8), jnp.float32),
        pltpu.VMEM((2, EB2, CW), jnp.float32),
        pltpu.VMEM_SHARED((NPAD, CW), jnp.float32),
        pltpu.SemaphoreType.DMA,
        pltpu.SemaphoreType.DMA,
        pltpu.SemaphoreType.DMA,
        pltpu.SemaphoreType.DMA,
    ],
)
def _sc_p2(big_r, w_e, src16_2, dst_i2, zeros_cw, out, gidx2, didx2, xb, wb,
           yb, num, g0, g1, w0, w1):
    c = lax.axis_index("c")
    s = lax.axis_index("s")
    gsems = (g0, g1)
    wsems = (w0, w1)

    lanes = lax.iota(jnp.int32, 16)

    pltpu.sync_copy(dst_i2.at[pl.ds(s * NB2, NB2)], didx2)

    for j in range(8):
        fidx = c * 8 + j
        pltpu.sync_copy(zeros_cw.at[pl.ds(s * ZR, ZR)], num.at[pl.ds(s * ZR, ZR)])
        pltpu.sync_copy(src16_2.at[fidx, pl.ds(s * NB2, NB2)], gidx2)
        plsc.subcore_barrier()

        def copies(u, b):
            ebase = s * (PADE // 16) + b * EB2
            return [
                pltpu.make_async_copy(big_r.at[gidx2.at[b]], xb.at[u], gsems[u]),
                pltpu.make_async_copy(w_e.at[pl.ds(ebase, EB2)], wb.at[u], wsems[u]),
            ]

        def issue(u, b):
            for cp in copies(u, b):
                cp.start()

        def wait(u, b):
            for cp in copies(u, b):
                cp.wait()

        hvec = jnp.full((16,), fidx >> 1, jnp.int32)

        def compute(u, b):
            def edge(e2, carry2):
                for d in range(2):
                    e = 2 * e2 + d
                    ev = jnp.full((16,), e, jnp.int32)
                    wv = plsc.load_gather(wb.at[u], [ev, hvec])
                    for k in range(CW // 16):
                        yb.at[u][e, pl.ds(k * 16, 16)] = (
                            xb.at[u][e, pl.ds(k * 16, 16)] * wv)
                return carry2

            lax.fori_loop(0, EB2 // 2, edge, 0)
            pltpu.sync_copy(yb.at[u], num.at[didx2.at[b]], add=True)

        issue(0, 0)

        def step(i, carry):
            b0 = 2 * i
            b1 = b0 + 1
            b2 = jnp.minimum(b0 + 2, NB2 - 1)
            issue(1, b1)
            wait(0, b0)
            compute(0, b0)
            issue(0, b2)
            wait(1, b1)
            compute(1, b1)
            return carry

        lax.fori_loop(0, NB2 // 2, step, 0)
        wait(0, NB2 - 1)
        plsc.subcore_barrier()
        pltpu.sync_copy(num.at[pl.ds(s * ZR, ZR)], out.at[fidx, pl.ds(s * ZR, ZR)])
        plsc.subcore_barrier()


def _tc_c_body(np_ref, dp_ref, b2_ref, o_ref):
    deni = 1.0 / (dp_ref[0] + dp_ref[1] + 1e-16)
    rb = deni.shape[0]
    acc = jnp.zeros((rb, 128), jnp.float32)
    for h in range(8):
        part = jnp.concatenate([np_ref[2 * h], np_ref[2 * h + 1]], axis=1)
        acc = acc + part * deni[:, h:h + 1]
    o = acc * 0.125 + b2_ref[...]
    m = jnp.max(o, axis=-1, keepdims=True)
    o_ref[...] = o - (m + jnp.log(jnp.sum(jnp.exp(o - m), axis=-1, keepdims=True)))


def _tc_c(numparts, denparts, b2r):
    rb = NPAD // 16
    return pl.pallas_call(
        _tc_c_body,
        grid=(16,),
        in_specs=[
            pl.BlockSpec((16, rb, CW), lambda i: (0, i, 0)),
            pl.BlockSpec((2, rb, 8), lambda i: (0, i, 0)),
            pl.BlockSpec((1, 128), lambda i: (0, 0)),
        ],
        out_specs=pl.BlockSpec((rb, 128), lambda i: (i, 0)),
        out_shape=jax.ShapeDtypeStruct((NPAD, 128), jnp.float32),
    )(numparts, denparts, b2r)


def kernel(x, edge_index, W1, att_l1, att_r1, b1, W2, att_l2, att_r2, b2):
    ei = edge_index.astype(jnp.int32)
    loop = jnp.arange(_N, dtype=jnp.int32)
    src = jnp.concatenate([ei[0], loop])
    dst = jnp.concatenate([ei[1], loop])
    npad_e = PADE - src.shape[0]
    src = jnp.concatenate([src, jnp.full((npad_e,), _N, jnp.int32)])
    dst = jnp.concatenate([dst, jnp.full((npad_e,), _N, jnp.int32)])

    w1r = W1.reshape(_D, _H, _C1)
    wcat1 = jnp.concatenate(
        [W1,
         jnp.einsum("dhc,hc->dh", w1r, att_l1[0]),
         jnp.einsum("dhc,hc->dh", w1r, att_r1[0])], axis=1)

    jcol = np.arange(_H * _C2)
    fj, hj, ccj = jcol // 64, (jcol % 64) // 8, jcol % 8
    w2s = W2[:, hj * _C2 + fj * 8 + ccj]
    w2r = W2.reshape(_H * _C1, _H, _C2)
    wlr2 = jnp.concatenate(
        [jnp.einsum("dhc,hc->dh", w2r, att_l2[0]),
         jnp.einsum("dhc,hc->dh", w2r, att_r2[0])], axis=1)

    xp = jnp.pad(x, ((0, NPAD - _N), (0, 0)))

    rows1 = _tc_a(xp, wcat1)
    parts1 = _sc_l1(rows1, src, dst, jnp.zeros((NPAD, 80), jnp.float32))
    big1, big2, alar2 = _tc_b(parts1, w2s, W2, wlr2, b1.reshape(1, 64))
    w_e, denparts = _sc_p1(big1, alar2, src.reshape(-1, EB1), dst.reshape(-1, EB1),
                           jnp.zeros((NPAD, 8), jnp.float32))
    src16_2 = (src[None, :] * 16 + jnp.arange(16, dtype=jnp.int32)[:, None]
               ).reshape(16, -1, EB2)
    numparts = _sc_p2(big2.reshape(NPAD * 16, CW), w_e, src16_2,
                      dst.reshape(-1, EB2), jnp.zeros((NPAD, CW), jnp.float32))
    out = _tc_c(numparts, denparts, b2.reshape(1, 128))
    return out[:_N], jnp.zeros(())
